# baseline (speedup 1.0000x reference)
import functools

import jax
import jax.numpy as jnp
from jax import lax
from jax.experimental import pallas as pl
from jax.experimental.pallas import tpu as pltpu
from jax.experimental.pallas import tpu_sc as plsc

_L = 16


def _sc_gather(H, V, S, NC, NS):
    NW = NC * NS
    rows_per_w = S // NW
    RB = 8
    n_ib = rows_per_w // RB
    HP = H // 2
    n_chunks = n_ib * HP

    mesh = plsc.VectorSubcoreMesh(core_axis_name="c", subcore_axis_name="s")

    @functools.partial(
        pl.kernel,
        out_type=jax.ShapeDtypeStruct((H, S, S), jnp.float32),
        mesh=mesh,
        scratch_types=[
            pltpu.VMEM((H * V,), jnp.float32),
            pltpu.VMEM((2 * RB, S), jnp.int32),
            pltpu.VMEM((4 * RB, S), jnp.float32),
            pltpu.SemaphoreType.DMA((2,)),
            pltpu.SemaphoreType.DMA((2,)),
        ],
        compiler_params=pltpu.CompilerParams(
            needs_layout_passes=False, use_tc_tiling_on_sc=True),
    )
    def run(table_hbm, idx_hbm, out_hbm, table_v, idx_v, out_v, semi, semo):
        wid = lax.axis_index("s") * NC + lax.axis_index("c")
        base_i = wid * rows_per_w
        pltpu.sync_copy(table_hbm, table_v)
        pltpu.async_copy(
            idx_hbm.at[pl.ds(base_i, RB), :],
            idx_v.at[pl.ds(0, RB), :], semi.at[0])

        def chunk_body(t, _):
            ib = t // HP
            hp = t - ib * HP
            pi = lax.rem(ib, 2)
            po = lax.rem(t, 2)
            h0 = 2 * hp
            i0 = base_i + ib * RB

            @pl.when(jnp.logical_and(hp == 0, ib + 1 < n_ib))
            def _():
                pltpu.async_copy(
                    idx_hbm.at[pl.ds(i0 + RB, RB), :],
                    idx_v.at[pl.ds((1 - pi) * RB, RB), :], semi.at[1 - pi])

            @pl.when(hp == 0)
            def _():
                pltpu.make_async_copy(
                    idx_hbm.at[pl.ds(i0, RB), :],
                    idx_v.at[pl.ds(pi * RB, RB), :], semi.at[pi]).wait()

            @pl.when(t >= 2)
            def _():
                for _hh in range(2):
                    pltpu.make_async_copy(
                        out_v.at[pl.ds(po * 2 * RB + _hh * RB, RB), :],
                        out_hbm.at[h0 + _hh, pl.ds(i0, RB), :],
                        semo.at[po]).wait()

            ir = pi * RB
            orow = po * 2 * RB
            for r in range(RB):
                @plsc.parallel_loop(0, S // _L, unroll=16)
                def _(n):
                    iv = idx_v[ir + r, pl.ds(n * _L, _L)]
                    out_v[orow + r, pl.ds(n * _L, _L)] = plsc.load_gather(
                        table_v, [h0 * V + iv])
                    out_v[orow + RB + r, pl.ds(n * _L, _L)] = (
                        plsc.load_gather(table_v, [(h0 + 1) * V + iv]))

            for hh in range(2):
                pltpu.async_copy(
                    out_v.at[pl.ds(orow + hh * RB, RB), :],
                    out_hbm.at[h0 + hh, pl.ds(i0, RB), :], semo.at[po])
            return 0

        lax.fori_loop(0, n_chunks, chunk_body, 0)

        for po in range(2):
            for hh in range(2):
                pltpu.make_async_copy(
                    out_v.at[pl.ds(po * 2 * RB + hh * RB, RB), :],
                    out_hbm.at[hh, pl.ds(base_i, RB), :], semo.at[po]).wait()

    return run


def kernel(relative_position_bias_table, relative_position_index):
    V, H = relative_position_bias_table.shape
    S = relative_position_index.shape[0]
    info = plsc.get_sparse_core_info()
    NC, NS = info.num_cores, info.num_subcores

    tableT = relative_position_bias_table.T.reshape(-1)

    return _sc_gather(H, V, S, NC, NS)(tableT, relative_position_index)


# speedup vs baseline: 1.0203x; 1.0203x over previous
import functools

import jax
import jax.numpy as jnp
from jax import lax
from jax.experimental import pallas as pl
from jax.experimental.pallas import tpu as pltpu
from jax.experimental.pallas import tpu_sc as plsc

_L = 16


def _sc_gather(H, V, S, NC, NS):
    NW = NC * NS
    rows_per_w = S // NW
    RB = 8
    n_ib = rows_per_w // RB
    HP = H // 2
    n_chunks = n_ib * HP

    mesh = plsc.VectorSubcoreMesh(core_axis_name="c", subcore_axis_name="s")

    @functools.partial(
        pl.kernel,
        out_type=jax.ShapeDtypeStruct((H, S, S), jnp.float32),
        mesh=mesh,
        scratch_types=[
            pltpu.VMEM((H * V,), jnp.float32),
            pltpu.VMEM((2 * RB, S), jnp.int32),
            pltpu.VMEM((6 * RB, S), jnp.float32),
            pltpu.SemaphoreType.DMA((2,)),
            pltpu.SemaphoreType.DMA((2,)),
        ],
        compiler_params=pltpu.CompilerParams(
            needs_layout_passes=False, use_tc_tiling_on_sc=True),
    )
    def run(table_hbm, idx_hbm, out_hbm, table_v, idx_v, out_v, semi, semo):
        wid = lax.axis_index("s") * NC + lax.axis_index("c")
        base_i = wid * rows_per_w
        pltpu.async_copy(
            idx_hbm.at[pl.ds(base_i, RB), :],
            idx_v.at[pl.ds(0, RB), :], semi.at[0])
        pltpu.sync_copy(table_hbm, table_v)

        def chunk_body(t, _):
            ib = t // HP
            hp = t - ib * HP
            pi = lax.rem(ib, 2)
            po = lax.rem(t, 3)
            h0 = 2 * hp
            i0 = base_i + ib * RB

            @pl.when(jnp.logical_and(hp == 0, ib + 1 < n_ib))
            def _():
                pltpu.async_copy(
                    idx_hbm.at[pl.ds(i0 + RB, RB), :],
                    idx_v.at[pl.ds((1 - pi) * RB, RB), :], semi.at[1 - pi])

            @pl.when(hp == 0)
            def _():
                pltpu.make_async_copy(
                    idx_hbm.at[pl.ds(i0, RB), :],
                    idx_v.at[pl.ds(pi * RB, RB), :], semi.at[pi]).wait()

            @pl.when(t >= 3)
            def _():
                for _hh in range(2):
                    pltpu.make_async_copy(
                        out_v.at[pl.ds(po * 2 * RB + _hh * RB, RB), :],
                        out_hbm.at[h0 + _hh, pl.ds(i0, RB), :],
                        semo.at[po]).wait()

            ir = pi * RB
            orow = po * 2 * RB
            for r in range(RB):
                @plsc.parallel_loop(0, S // _L, unroll=8)
                def _(n):
                    iv = idx_v[ir + r, pl.ds(n * _L, _L)]
                    out_v[orow + r, pl.ds(n * _L, _L)] = plsc.load_gather(
                        table_v, [h0 * V + iv])
                    out_v[orow + RB + r, pl.ds(n * _L, _L)] = (
                        plsc.load_gather(table_v, [(h0 + 1) * V + iv]))

            for hh in range(2):
                pltpu.async_copy(
                    out_v.at[pl.ds(orow + hh * RB, RB), :],
                    out_hbm.at[h0 + hh, pl.ds(i0, RB), :], semo.at[po])
            return 0

        lax.fori_loop(0, n_chunks, chunk_body, 0)

        for po in range(3):
            for hh in range(2):
                pltpu.make_async_copy(
                    out_v.at[pl.ds(po * 2 * RB + hh * RB, RB), :],
                    out_hbm.at[hh, pl.ds(base_i, RB), :], semo.at[po]).wait()

    return run


def kernel(relative_position_bias_table, relative_position_index):
    V, H = relative_position_bias_table.shape
    S = relative_position_index.shape[0]
    info = plsc.get_sparse_core_info()
    NC, NS = info.num_cores, info.num_subcores

    tableT = relative_position_bias_table.T.reshape(-1)

    return _sc_gather(H, V, S, NC, NS)(tableT, relative_position_index)


# speedup vs baseline: 1.1528x; 1.1299x over previous
import functools

import jax
import jax.numpy as jnp
from jax import lax
from jax.experimental import pallas as pl
from jax.experimental.pallas import tpu as pltpu
from jax.experimental.pallas import tpu_sc as plsc

_L = 16
_HQ = 4


def _sc_gather(H, V, S, NC, NS):
    NW = NC * NS
    HG = H // 2
    n_rw = NW // 2
    rows_per_w = S // n_rw
    RB = 8
    n_ib = rows_per_w // RB
    NQ = HG // _HQ
    n_chunks = n_ib * NQ

    mesh = plsc.VectorSubcoreMesh(core_axis_name="c", subcore_axis_name="s")

    @functools.partial(
        pl.kernel,
        out_type=jax.ShapeDtypeStruct((H, S, S), jnp.float32),
        mesh=mesh,
        scratch_types=[
            pltpu.VMEM((HG * V,), jnp.float32),
            pltpu.VMEM((2 * RB, S), jnp.int32),
            pltpu.VMEM((2 * _HQ * RB, S), jnp.float32),
            pltpu.SemaphoreType.DMA((2,)),
            pltpu.SemaphoreType.DMA((2,)),
        ],
        compiler_params=pltpu.CompilerParams(
            needs_layout_passes=False, use_tc_tiling_on_sc=True),
    )
    def run(table_hbm, idx_hbm, out_hbm, table_v, idx_v, out_v, semi, semo):
        wid = lax.axis_index("s") * NC + lax.axis_index("c")
        hg = wid // n_rw
        rw = lax.rem(wid, n_rw)
        hbase = hg * HG
        base_i = rw * rows_per_w
        pltpu.async_copy(
            idx_hbm.at[pl.ds(base_i, RB), :],
            idx_v.at[pl.ds(0, RB), :], semi.at[0])
        pltpu.sync_copy(table_hbm.at[pl.ds(hbase * V, HG * V)], table_v)

        def chunk_body(t, _):
            ib = t // NQ
            hq = t - ib * NQ
            pi = lax.rem(ib, 2)
            po = lax.rem(t, 2)
            h0 = hbase + hq * _HQ
            l0 = hq * _HQ
            i0 = base_i + ib * RB

            @pl.when(jnp.logical_and(hq == 0, ib + 1 < n_ib))
            def _():
                pltpu.async_copy(
                    idx_hbm.at[pl.ds(i0 + RB, RB), :],
                    idx_v.at[pl.ds((1 - pi) * RB, RB), :], semi.at[1 - pi])

            @pl.when(hq == 0)
            def _():
                pltpu.make_async_copy(
                    idx_hbm.at[pl.ds(i0, RB), :],
                    idx_v.at[pl.ds(pi * RB, RB), :], semi.at[pi]).wait()

            @pl.when(t >= 2)
            def _():
                for _q in range(_HQ):
                    pltpu.make_async_copy(
                        out_v.at[pl.ds((po * _HQ + _q) * RB, RB), :],
                        out_hbm.at[h0 + _q, pl.ds(i0, RB), :],
                        semo.at[po]).wait()

            ir = pi * RB
            orow = po * _HQ * RB
            for r in range(RB):
                @plsc.parallel_loop(0, S // _L, unroll=8)
                def _(n):
                    iv = idx_v[ir + r, pl.ds(n * _L, _L)]
                    for q in range(_HQ):
                        out_v[orow + q * RB + r, pl.ds(n * _L, _L)] = (
                            plsc.load_gather(table_v, [(l0 + q) * V + iv]))

            for q in range(_HQ):
                pltpu.async_copy(
                    out_v.at[pl.ds(orow + q * RB, RB), :],
                    out_hbm.at[h0 + q, pl.ds(i0, RB), :], semo.at[po])
            return 0

        lax.fori_loop(0, n_chunks, chunk_body, 0)

        for po in range(2):
            for q in range(_HQ):
                pltpu.make_async_copy(
                    out_v.at[pl.ds((po * _HQ + q) * RB, RB), :],
                    out_hbm.at[hbase + q, pl.ds(base_i, RB), :],
                    semo.at[po]).wait()

    return run


def kernel(relative_position_bias_table, relative_position_index):
    V, H = relative_position_bias_table.shape
    S = relative_position_index.shape[0]
    info = plsc.get_sparse_core_info()
    NC, NS = info.num_cores, info.num_subcores

    tableT = relative_position_bias_table.T.reshape(-1)

    return _sc_gather(H, V, S, NC, NS)(tableT, relative_position_index)
